# baseline (speedup 1.0000x reference)
import functools

import jax
import jax.numpy as jnp
from jax import lax
from jax.experimental import pallas as pl
from jax.experimental.pallas import tpu as pltpu
from jax.experimental.pallas import tpu_sc as plsc


def _sc_gather(table, idx):
    (n,) = idx.shape
    info = plsc.get_sparse_core_info()
    nw = info.num_cores * info.num_subcores
    b_per_w = n // nw
    mesh = plsc.VectorSubcoreMesh(core_axis_name="c", subcore_axis_name="s")

    @functools.partial(
        pl.kernel,
        mesh=mesh,
        out_type=jax.ShapeDtypeStruct((n,), jnp.float32),
        scratch_types=[
            pltpu.VMEM((b_per_w,), jnp.int32),
            pltpu.VMEM((b_per_w,), jnp.float32),
            pltpu.SemaphoreType.DMA,
        ],
    )
    def k(table_hbm, idx_hbm, out_hbm, idx_v, rows_v, sem):
        wid = lax.axis_index("s") * info.num_cores + lax.axis_index("c")
        base = wid * b_per_w
        pltpu.sync_copy(idx_hbm.at[pl.ds(base, b_per_w)], idx_v)
        pltpu.async_copy(table_hbm.at[idx_v], rows_v, sem).wait()
        pltpu.sync_copy(rows_v, out_hbm.at[pl.ds(base, b_per_w)])

    return k(table, idx)


def _dense_body(u_ref, p_ref, t_ref, w1_ref, w2_ref, o_ref):
    t = t_ref[...]
    d1 = u_ref[...] - t
    d2 = p_ref[...] - t
    sq = d1 * d1 * w1_ref[...] + d2 * d2 * w2_ref[...]
    o_ref[...] = jnp.sum(sq.T, axis=0, keepdims=True)[None]


def _add_body(a_ref, b_ref, o_ref):
    o_ref[...] = a_ref[...] + b_ref[...]


def kernel(user_ebs, playlist_ebs, track_ebs, track_entity_ids, B1, B2, track_biases):
    batch, eb = user_ebs.shape
    bias = _sc_gather(track_biases, track_entity_ids.astype(jnp.int32))

    grid = 8
    blk = batch // grid
    w1 = (B1 * B1).reshape(1, eb)
    w2 = (B2 * B2).reshape(1, eb)
    o12 = pl.pallas_call(
        _dense_body,
        grid=(grid,),
        in_specs=[
            pl.BlockSpec((blk, eb), lambda i: (i, 0)),
            pl.BlockSpec((blk, eb), lambda i: (i, 0)),
            pl.BlockSpec((blk, eb), lambda i: (i, 0)),
            pl.BlockSpec((1, eb), lambda i: (0, 0)),
            pl.BlockSpec((1, eb), lambda i: (0, 0)),
        ],
        out_specs=pl.BlockSpec((1, 1, blk), lambda i: (i, 0, 0)),
        out_shape=jax.ShapeDtypeStruct((grid, 1, blk), jnp.float32),
    )(user_ebs, playlist_ebs, track_ebs, w1, w2)
    o12 = o12.reshape(grid, blk)

    out2d = pl.pallas_call(
        _add_body,
        in_specs=[
            pl.BlockSpec((grid, blk), lambda: (0, 0)),
            pl.BlockSpec((grid, blk), lambda: (0, 0)),
        ],
        out_specs=pl.BlockSpec((grid, blk), lambda: (0, 0)),
        out_shape=jax.ShapeDtypeStruct((grid, blk), jnp.float32),
    )(o12, bias.reshape(grid, blk))
    return out2d.reshape(batch)


# speedup vs baseline: 1.7417x; 1.7417x over previous
import functools

import jax
import jax.numpy as jnp
from jax import lax
from jax.experimental import pallas as pl
from jax.experimental.pallas import tpu as pltpu
from jax.experimental.pallas import tpu_sc as plsc


def _sc_gather(table, idx):
    (n,) = idx.shape
    info = plsc.get_sparse_core_info()
    nw = info.num_cores * info.num_subcores
    b_per_w = n // nw
    mesh = plsc.VectorSubcoreMesh(core_axis_name="c", subcore_axis_name="s")

    @functools.partial(
        pl.kernel,
        mesh=mesh,
        out_type=jax.ShapeDtypeStruct((n,), jnp.float32),
        scratch_types=[
            pltpu.VMEM((b_per_w,), jnp.int32),
            pltpu.VMEM((b_per_w,), jnp.float32),
            pltpu.SemaphoreType.DMA,
        ],
    )
    def k(table_hbm, idx_hbm, out_hbm, idx_v, rows_v, sem):
        wid = lax.axis_index("s") * info.num_cores + lax.axis_index("c")
        base = wid * b_per_w
        pltpu.sync_copy(idx_hbm.at[pl.ds(base, b_per_w)], idx_v)
        pltpu.async_copy(table_hbm.at[idx_v], rows_v, sem).wait()
        pltpu.sync_copy(rows_v, out_hbm.at[pl.ds(base, b_per_w)])

    return k(table, idx)


def _dense_body(u_ref, p_ref, t_ref, b1_ref, b2_ref, bias_ref, o_ref):
    t = t_ref[...]
    d1 = (u_ref[...] - t) * b1_ref[...]
    d2 = (p_ref[...] - t) * b2_ref[...]
    sq = d1 * d1 + d2 * d2
    o_ref[...] = jnp.sum(sq, axis=0) + bias_ref[...]


def kernel(user_ebs, playlist_ebs, track_ebs, track_entity_ids, B1, B2, track_biases):
    batch, eb = user_ebs.shape
    bias = _sc_gather(track_biases, track_entity_ids.astype(jnp.int32))

    grid = 8
    blk = batch // grid
    out = pl.pallas_call(
        _dense_body,
        grid=(grid,),
        in_specs=[
            pl.BlockSpec((eb, blk), lambda i: (0, i)),
            pl.BlockSpec((eb, blk), lambda i: (0, i)),
            pl.BlockSpec((eb, blk), lambda i: (0, i)),
            pl.BlockSpec((eb, 1), lambda i: (0, 0)),
            pl.BlockSpec((eb, 1), lambda i: (0, 0)),
            pl.BlockSpec((blk,), lambda i: (i,)),
        ],
        out_specs=pl.BlockSpec((blk,), lambda i: (i,)),
        out_shape=jax.ShapeDtypeStruct((batch,), jnp.float32),
    )(user_ebs.T, playlist_ebs.T, track_ebs.T,
      B1.reshape(eb, 1), B2.reshape(eb, 1), bias)
    return out


# speedup vs baseline: 1.9417x; 1.1148x over previous
import functools

import jax
import jax.numpy as jnp
from jax import lax
from jax.experimental import pallas as pl
from jax.experimental.pallas import tpu as pltpu
from jax.experimental.pallas import tpu_sc as plsc


def _sc_gather(table, idx):
    (n,) = idx.shape
    info = plsc.get_sparse_core_info()
    nw = info.num_subcores
    b_per_w = n // nw
    mesh = plsc.VectorSubcoreMesh(
        core_axis_name="c", subcore_axis_name="s", num_cores=1)

    @functools.partial(
        pl.kernel,
        mesh=mesh,
        out_type=jax.ShapeDtypeStruct((n,), jnp.float32),
        scratch_types=[
            pltpu.VMEM((b_per_w,), jnp.int32),
            pltpu.VMEM((b_per_w,), jnp.float32),
            pltpu.SemaphoreType.DMA,
        ],
    )
    def k(table_hbm, idx_hbm, out_hbm, idx_v, rows_v, sem):
        wid = lax.axis_index("s")
        base = wid * b_per_w
        pltpu.sync_copy(idx_hbm.at[pl.ds(base, b_per_w)], idx_v)
        pltpu.async_copy(table_hbm.at[idx_v], rows_v, sem).wait()
        pltpu.sync_copy(rows_v, out_hbm.at[pl.ds(base, b_per_w)])

    return k(table, idx)


def _dense_body(u_ref, p_ref, t_ref, w1_ref, w2_ref, o_ref):
    t = t_ref[...]
    d1 = u_ref[...] - t
    d2 = p_ref[...] - t
    sq = d1 * d1 * w1_ref[...] + d2 * d2 * w2_ref[...]
    partial = jnp.sum(sq, axis=0)

    @pl.when(pl.program_id(0) == 0)
    def _():
        o_ref[...] = partial

    @pl.when(pl.program_id(0) != 0)
    def _():
        o_ref[...] += partial


def _add_body(a_ref, b_ref, o_ref):
    o_ref[...] = a_ref[...] + b_ref[...]


def kernel(user_ebs, playlist_ebs, track_ebs, track_entity_ids, B1, B2, track_biases):
    batch, eb = user_ebs.shape
    bias = _sc_gather(track_biases, track_entity_ids.astype(jnp.int32))

    grid = 8
    rows = eb // grid
    w1 = (B1 * B1).reshape(eb, 1)
    w2 = (B2 * B2).reshape(eb, 1)
    o12 = pl.pallas_call(
        _dense_body,
        grid=(grid,),
        in_specs=[
            pl.BlockSpec((rows, batch), lambda i: (i, 0)),
            pl.BlockSpec((rows, batch), lambda i: (i, 0)),
            pl.BlockSpec((rows, batch), lambda i: (i, 0)),
            pl.BlockSpec((rows, 1), lambda i: (i, 0)),
            pl.BlockSpec((rows, 1), lambda i: (i, 0)),
        ],
        out_specs=pl.BlockSpec((batch,), lambda i: (0,)),
        out_shape=jax.ShapeDtypeStruct((batch,), jnp.float32),
    )(user_ebs.T, playlist_ebs.T, track_ebs.T, w1, w2)

    out = pl.pallas_call(
        _add_body,
        in_specs=[
            pl.BlockSpec((batch,), lambda: (0,)),
            pl.BlockSpec((batch,), lambda: (0,)),
        ],
        out_specs=pl.BlockSpec((batch,), lambda: (0,)),
        out_shape=jax.ShapeDtypeStruct((batch,), jnp.float32),
    )(o12, bias)
    return out
